# Initial kernel scaffold; baseline (speedup 1.0000x reference)
import jax
import jax.numpy as jnp
from jax.experimental import pallas as pl


def kernel(x, indices, values, bias):
    raise NotImplementedError("write your pallas kernel here")


# speedup vs baseline: 3.3345x; 3.3345x over previous
import functools

import jax
import jax.numpy as jnp
from jax.experimental import pallas as pl
from jax.experimental.pallas import tpu as pltpu

N = 4096
M = 4096
E = 64
BB = 256


def _tile_body(idx_ref, val_ref, bias_ref, x_ref, out_ref):
    src = idx_ref[0:1, :]
    dst = idx_ref[1:2, :]
    vals = val_ref[0:1, :]

    n_iota = jax.lax.broadcasted_iota(jnp.int32, (N, E), 0)
    g = (n_iota == src).astype(jnp.float32)
    xg = jax.lax.dot_general(
        x_ref[...], g,
        dimension_numbers=(((1,), (0,)), ((), ())),
        preferred_element_type=jnp.float32,
    )

    m_iota = jax.lax.broadcasted_iota(jnp.int32, (M, E), 0)
    s_t = jnp.where(m_iota == dst, vals, 0.0)
    out = jax.lax.dot_general(
        xg, s_t,
        dimension_numbers=(((1,), (1,)), ((), ())),
        preferred_element_type=jnp.float32,
    )
    out_ref[...] = out + bias_ref[...]


@functools.partial(jax.jit, static_argnames=("interpret",))
def kernel(x, indices, values, bias, interpret=False):
    b = x.shape[0]
    x2 = x.reshape(b, N)
    vals2 = values.reshape(1, E)
    bias2 = bias.reshape(1, M)
    grid = (b // BB,)
    out = pl.pallas_call(
        _tile_body,
        grid=grid,
        in_specs=[
            pl.BlockSpec((2, E), lambda i: (0, 0)),
            pl.BlockSpec((1, E), lambda i: (0, 0)),
            pl.BlockSpec((1, M), lambda i: (0, 0)),
            pl.BlockSpec((BB, N), lambda i: (i, 0)),
        ],
        out_specs=pl.BlockSpec((BB, M), lambda i: (i, 0)),
        out_shape=jax.ShapeDtypeStruct((b, M), jnp.float32),
        interpret=interpret,
    )(indices, vals2, bias2, x2)
    return out.reshape(b, M, 1)


# speedup vs baseline: 5.4026x; 1.6202x over previous
import functools

import jax
import jax.numpy as jnp
from jax import lax
from jax.experimental import pallas as pl
from jax.experimental.pallas import tpu as pltpu
from jax.experimental.pallas import tpu_sc as plsc

N = 4096
M = 4096
E = 64
BB = 256

_SC_INFO = plsc.get_sparse_core_info()
_NC = _SC_INFO.num_cores
_NS = _SC_INFO.num_subcores
_NW = _NC * _NS


def _make_sc_gather(total):
    per_w = total // _NW
    mesh = plsc.VectorSubcoreMesh(core_axis_name="c", subcore_axis_name="s")

    @functools.partial(
        pl.kernel,
        mesh=mesh,
        out_type=jax.ShapeDtypeStruct((total,), jnp.float32),
        scratch_types=[
            pltpu.VMEM((per_w,), jnp.int32),
            pltpu.VMEM((per_w,), jnp.float32),
            pltpu.SemaphoreType.DMA,
        ],
    )
    def gather_k(xflat_hbm, idx_hbm, out_hbm, idx_v, val_v, sem):
        wid = lax.axis_index("s") * _NC + lax.axis_index("c")
        base = wid * per_w
        pltpu.sync_copy(idx_hbm.at[pl.ds(base, per_w)], idx_v)
        pltpu.async_copy(xflat_hbm.at[idx_v], val_v, sem).wait()
        pltpu.sync_copy(val_v, out_hbm.at[pl.ds(base, per_w)])

    return gather_k


def _tile_body(idx_ref, val_ref, bias_ref, xg_ref, out_ref):
    dst = idx_ref[1:2, :]
    vals = val_ref[0:1, :]
    m_iota = jax.lax.broadcasted_iota(jnp.int32, (M, E), 0)
    s_t = jnp.where(m_iota == dst, vals, 0.0)
    out = jax.lax.dot_general(
        xg_ref[...], s_t,
        dimension_numbers=(((1,), (1,)), ((), ())),
        preferred_element_type=jnp.float32,
    )
    out_ref[...] = out + bias_ref[...]


@jax.jit
def kernel(x, indices, values, bias):
    b = x.shape[0]
    xflat = x.reshape(b * N)
    flat_idx = (
        jnp.arange(b, dtype=jnp.int32)[:, None] * N + indices[0][None, :]
    ).reshape(b * E)
    xg = _make_sc_gather(b * E)(xflat, flat_idx).reshape(b, E)

    vals2 = values.reshape(1, E)
    bias2 = bias.reshape(1, M)
    out = pl.pallas_call(
        _tile_body,
        grid=(b // BB,),
        in_specs=[
            pl.BlockSpec((2, E), lambda i: (0, 0)),
            pl.BlockSpec((1, E), lambda i: (0, 0)),
            pl.BlockSpec((1, M), lambda i: (0, 0)),
            pl.BlockSpec((BB, E), lambda i: (i, 0)),
        ],
        out_specs=pl.BlockSpec((BB, M), lambda i: (i, 0)),
        out_shape=jax.ShapeDtypeStruct((b, M), jnp.float32),
    )(indices, vals2, bias2, xg)
    return out.reshape(b, M, 1)
